# baseline (speedup 1.0000x reference)
import functools

import jax
import jax.numpy as jnp
from jax import lax
from jax.experimental import pallas as pl
from jax.experimental.pallas import tpu as pltpu
from jax.experimental.pallas import tpu_sc as plsc

_D = 64
_NF = 26
_B = 4096
_NW = 32
_CH = 128
_NSL = 16

_DIN = _NF * _D
_NCT = _DIN // 128
_BT = 2048


def _sc_gather(x_i32, table, half, nb):
    ch = nb // _NW
    nsl = ch // 8
    mesh = plsc.VectorSubcoreMesh(core_axis_name="c", subcore_axis_name="s")

    @functools.partial(
        pl.kernel,
        mesh=mesh,
        compiler_params=pltpu.CompilerParams(use_tc_tiling_on_sc=False),
        out_type=jax.ShapeDtypeStruct((nb // 8, _NCT, 8, 128), jnp.float32),
        scratch_types=[
            pltpu.VMEM((_NF, ch), jnp.int32),
            pltpu.VMEM((_NF, ch), jnp.int32),
            pltpu.VMEM((2, ch, _D), jnp.float32),
            pltpu.SemaphoreType.DMA,
            pltpu.SemaphoreType.DMA,
            pltpu.SemaphoreType.DMA,
            pltpu.SemaphoreType.DMA,
        ],
    )
    def gather_k(x_hbm, table_hbm, out_hbm, xblk, gidx, rows_v,
                 sem0, sem1, psem0, psem1):
        wid = lax.axis_index("s") * 2 + lax.axis_index("c")
        s0 = wid * nsl

        for f in range(_NF):
            pltpu.async_copy(x_hbm.at[f, pl.ds(half * nb + wid * ch, ch)],
                             xblk.at[f], sem0)
        for f in range(_NF):
            pltpu.make_async_copy(x_hbm.at[f, pl.ds(0, ch)],
                                  xblk.at[f], sem0).wait()

        for f in range(_NF):
            for v8 in range(ch // 16):
                gidx[f, pl.ds(v8 * 16, 16)] = (
                    xblk[f, pl.ds(v8 * 16, 16)] * 2)

        def fire(f, slot, sem):
            pltpu.async_copy(table_hbm.at[gidx.at[f]], rows_v.at[slot], sem)

        def drain(slot, sem):
            pltpu.make_async_copy(
                table_hbm.at[pl.ds(0, ch)], rows_v.at[slot], sem
            ).wait()

        def put(ct, h, slot, psem):
            for sl in range(nsl):
                pltpu.async_copy(
                    rows_v.at[slot, pl.ds(sl * 8, 8)],
                    out_hbm.at[s0 + sl, ct, slice(None), pl.ds(h * _D, _D)],
                    psem)

        def drain_put(slot, psem):
            for sl in range(nsl):
                pltpu.make_async_copy(
                    rows_v.at[slot, pl.ds(sl * 8, 8)],
                    out_hbm.at[sl, 0, slice(None), pl.ds(0, _D)], psem).wait()

        fire(0, 0, sem0)

        def body(ct, carry):
            f0 = 2 * ct

            @pl.when(ct > 0)
            def _():
                drain_put(1, psem1)

            fire(f0 + 1, 1, sem1)
            drain(0, sem0)
            put(ct, 0, 0, psem0)
            drain_put(0, psem0)

            @pl.when(f0 + 2 < _NF)
            def _():
                fire(f0 + 2, 0, sem0)

            drain(1, sem1)
            put(ct, 1, 1, psem1)
            return carry

        lax.fori_loop(0, _NCT, body, 0)
        drain_put(1, psem1)

    return gather_k(x_i32, table)


_NHALF = 2


def _mlp_body(x_ref, w1_ref, b1_ref, w2_ref, b2_ref, w3_ref, b3_ref,
              wo_ref, bo_ref, o_ref, xs_ref):
    for ct in range(_NCT):
        xs_ref[:, ct * 128:(ct + 1) * 128] = x_ref[:, ct].reshape(_BT, 128)
    h = jnp.maximum(
        jnp.dot(xs_ref[...].astype(jnp.bfloat16),
                w1_ref[...].astype(jnp.bfloat16),
                preferred_element_type=jnp.float32) + b1_ref[...],
        0.0)
    h = jnp.maximum(
        jnp.dot(h.astype(jnp.bfloat16), w2_ref[...].astype(jnp.bfloat16),
                preferred_element_type=jnp.float32) + b2_ref[...],
        0.0)
    h = jnp.maximum(
        jnp.dot(h.astype(jnp.bfloat16), w3_ref[...].astype(jnp.bfloat16),
                preferred_element_type=jnp.float32) + b3_ref[...],
        0.0)
    logit = jnp.sum(h * wo_ref[...], axis=1, keepdims=True) + bo_ref[...]
    o_ref[...] = jax.nn.sigmoid(logit)


def _tc_mlp(x4d, W1, b1, W2, b2, W3, b3, Wo, bo, nb):
    d1, d2, d3 = W1.shape[1], W2.shape[1], W3.shape[1]
    rep = lambda shape: pl.BlockSpec(shape, lambda i: tuple(0 for _ in shape))
    return pl.pallas_call(
        _mlp_body,
        grid=(nb // _BT,),
        in_specs=[
            pl.BlockSpec((_BT // 8, _NCT, 8, 128), lambda i: (i, 0, 0, 0)),
            rep((_DIN, d1)), rep((1, d1)),
            rep((d1, d2)), rep((1, d2)),
            rep((d2, d3)), rep((1, d3)),
            rep((1, d3)), rep((1, 1)),
        ],
        out_specs=pl.BlockSpec((_BT, 1), lambda i: (i, 0)),
        out_shape=jax.ShapeDtypeStruct((nb, 1), jnp.float32),
        scratch_shapes=[pltpu.VMEM((_BT, _DIN), jnp.float32)],
    )(x4d, W1, b1.reshape(1, d1), W2, b2.reshape(1, d2),
      W3, b3.reshape(1, d3), Wo.reshape(1, d3), bo.reshape(1, 1))


def kernel(x, emb, W1, b1, W2, b2, W3, b3, Wo, bo):
    emb128 = jnp.pad(emb, ((0, 0), (0, 64))).reshape(2 * emb.shape[0], _D)
    xi = x.astype(jnp.int32)
    x4d = _sc_gather(xi, emb128, 0, _B)
    return _tc_mlp(x4d, W1, b1, W2, b2, W3, b3, Wo, bo, _B)


# speedup vs baseline: 1.0415x; 1.0415x over previous
import functools

import jax
import jax.numpy as jnp
from jax import lax
from jax.experimental import pallas as pl
from jax.experimental.pallas import tpu as pltpu
from jax.experimental.pallas import tpu_sc as plsc

_D = 64
_NF = 26
_B = 4096
_NW = 32
_CH = 128
_NSL = 16

_DIN = _NF * _D
_NCT = _DIN // 128
_BT = 1024


def _sc_gather(x_i32, table, half, nb):
    ch = nb // _NW
    nsl = ch // 8
    mesh = plsc.VectorSubcoreMesh(core_axis_name="c", subcore_axis_name="s")

    @functools.partial(
        pl.kernel,
        mesh=mesh,
        compiler_params=pltpu.CompilerParams(use_tc_tiling_on_sc=False),
        out_type=jax.ShapeDtypeStruct((nb // 8, _NCT, 8, 128), jnp.float32),
        scratch_types=[
            pltpu.VMEM((_NF, ch), jnp.int32),
            pltpu.VMEM((_NF, ch), jnp.int32),
            pltpu.VMEM((4, ch, _D), jnp.float32),
            pltpu.SemaphoreType.DMA,
            pltpu.SemaphoreType.DMA,
            pltpu.SemaphoreType.DMA,
            pltpu.SemaphoreType.DMA,
            pltpu.SemaphoreType.DMA,
            pltpu.SemaphoreType.DMA,
            pltpu.SemaphoreType.DMA,
            pltpu.SemaphoreType.DMA,
        ],
    )
    def gather_k(x_hbm, table_hbm, out_hbm, xblk, gidx, rows_v,
                 sem0, sem1, sem2, sem3, psem0, psem1, psem2, psem3):
        wid = lax.axis_index("s") * 2 + lax.axis_index("c")
        s0 = wid * nsl

        for f in range(_NF):
            pltpu.async_copy(x_hbm.at[f, pl.ds(half * nb + wid * ch, ch)],
                             xblk.at[f], sem0)
        for f in range(_NF):
            pltpu.make_async_copy(x_hbm.at[f, pl.ds(0, ch)],
                                  xblk.at[f], sem0).wait()

        for f in range(_NF):
            for v8 in range(ch // 16):
                gidx[f, pl.ds(v8 * 16, 16)] = (
                    xblk[f, pl.ds(v8 * 16, 16)] * 2)

        def fire(f, slot, sem):
            pltpu.async_copy(table_hbm.at[gidx.at[f]], rows_v.at[slot], sem)

        def drain(slot, sem):
            pltpu.make_async_copy(
                table_hbm.at[pl.ds(0, ch)], rows_v.at[slot], sem
            ).wait()

        def put(ct, h, slot, psem):
            for sl in range(nsl):
                pltpu.async_copy(
                    rows_v.at[slot, pl.ds(sl * 8, 8)],
                    out_hbm.at[s0 + sl, ct, slice(None), pl.ds(h * _D, _D)],
                    psem)

        def drain_put(slot, psem):
            for sl in range(nsl):
                pltpu.make_async_copy(
                    rows_v.at[slot, pl.ds(sl * 8, 8)],
                    out_hbm.at[sl, 0, slice(None), pl.ds(0, _D)], psem).wait()

        fire(0, 0, sem0)
        fire(1, 1, sem1)

        def bank_body(ct, sa, sb, sema, semb, psa, psb, osa, osb,
                      osema, osemb, opsa, opsb):
            @pl.when(ct > 0)
            def _():
                drain_put(osa, opsa)
                drain_put(osb, opsb)

            @pl.when(ct + 1 < _NCT)
            def _():
                fire(2 * ct + 2, osa, osema)
                fire(2 * ct + 3, osb, osemb)

            drain(sa, sema)
            put(ct, 0, sa, psa)
            drain(sb, semb)
            put(ct, 1, sb, psb)

        def body(ct, carry):
            @pl.when(lax.rem(ct, 2) == 0)
            def _():
                bank_body(ct, 0, 1, sem0, sem1, psem0, psem1,
                          2, 3, sem2, sem3, psem2, psem3)

            @pl.when(lax.rem(ct, 2) == 1)
            def _():
                bank_body(ct, 2, 3, sem2, sem3, psem2, psem3,
                          0, 1, sem0, sem1, psem0, psem1)
            return carry

        lax.fori_loop(0, _NCT, body, 0)
        drain_put(0, psem0)
        drain_put(1, psem1)

    return gather_k(x_i32, table)


_NHALF = 2


def _mlp_body(x_ref, w1_ref, b1_ref, w2_ref, b2_ref, w3_ref, b3_ref,
              wo_ref, bo_ref, o_ref, xs_ref):
    for ct in range(_NCT):
        xs_ref[:, ct * 128:(ct + 1) * 128] = x_ref[:, ct].reshape(_BT, 128)
    h = jnp.maximum(
        jnp.dot(xs_ref[...].astype(jnp.bfloat16),
                w1_ref[...].astype(jnp.bfloat16),
                preferred_element_type=jnp.float32) + b1_ref[...],
        0.0)
    h = jnp.maximum(
        jnp.dot(h.astype(jnp.bfloat16), w2_ref[...].astype(jnp.bfloat16),
                preferred_element_type=jnp.float32) + b2_ref[...],
        0.0)
    h = jnp.maximum(
        jnp.dot(h.astype(jnp.bfloat16), w3_ref[...].astype(jnp.bfloat16),
                preferred_element_type=jnp.float32) + b3_ref[...],
        0.0)
    logit = jnp.sum(h * wo_ref[...], axis=1, keepdims=True) + bo_ref[...]
    o_ref[...] = jax.nn.sigmoid(logit)


def _tc_mlp(x4d, W1, b1, W2, b2, W3, b3, Wo, bo, nb):
    d1, d2, d3 = W1.shape[1], W2.shape[1], W3.shape[1]
    rep = lambda shape: pl.BlockSpec(shape, lambda i: tuple(0 for _ in shape))
    return pl.pallas_call(
        _mlp_body,
        grid=(nb // _BT,),
        in_specs=[
            pl.BlockSpec((_BT // 8, _NCT, 8, 128), lambda i: (i, 0, 0, 0)),
            rep((_DIN, d1)), rep((1, d1)),
            rep((d1, d2)), rep((1, d2)),
            rep((d2, d3)), rep((1, d3)),
            rep((1, d3)), rep((1, 1)),
        ],
        out_specs=pl.BlockSpec((_BT, 1), lambda i: (i, 0)),
        out_shape=jax.ShapeDtypeStruct((nb, 1), jnp.float32),
        scratch_shapes=[pltpu.VMEM((_BT, _DIN), jnp.float32)],
    )(x4d, W1, b1.reshape(1, d1), W2, b2.reshape(1, d2),
      W3, b3.reshape(1, d3), Wo.reshape(1, d3), bo.reshape(1, 1))


def kernel(x, emb, W1, b1, W2, b2, W3, b3, Wo, bo):
    emb128 = jnp.pad(emb, ((0, 0), (0, 64))).reshape(2 * emb.shape[0], _D)
    xi = x.astype(jnp.int32)
    x4d = _sc_gather(xi, emb128, 0, _B)
    return _tc_mlp(x4d, W1, b1, W2, b2, W3, b3, Wo, bo, _B)
